# baseline (speedup 1.0000x reference)
import functools
import jax
import jax.numpy as jnp
from jax import lax
from jax.experimental import pallas as pl
from jax.experimental.pallas import tpu as pltpu
from jax.experimental.pallas import tpu_sc as plsc

H = 4
N = 10000
C = 128
NPAD = 10112
NB = 128
HB = 400
NEG = -1e30


def _mm_body(x_ref, w_ref, u_ref, xw_ref, xu_ref):
    x = x_ref[...]
    xw_ref[...] = jnp.dot(x, w_ref[...], preferred_element_type=jnp.float32)
    xu_ref[...] = jnp.dot(x, u_ref[...], preferred_element_type=jnp.float32)


def _mm(x, W, U128):
    nblk = NPAD // NB
    return pl.pallas_call(
        _mm_body,
        grid=(nblk,),
        in_specs=[pl.BlockSpec((NB, C), lambda i: (i, 0)),
                  pl.BlockSpec((C, 4 * C), lambda i: (0, 0)),
                  pl.BlockSpec((C, C), lambda i: (0, 0))],
        out_specs=[pl.BlockSpec((NB, 4 * C), lambda i: (i, 0)),
                   pl.BlockSpec((NB, C), lambda i: (i, 0))],
        out_shape=[jax.ShapeDtypeStruct((NPAD, 4 * C), jnp.float32),
                   jax.ShapeDtypeStruct((NPAD, C), jnp.float32)],
    )(x, W, U128)


def _epi_body(agg_ref, xw_ref, deg_ref, cpad_ref, b_ref, out_ref):
    a = agg_ref[0] + agg_ref[1]
    deg = deg_ref[0] + deg_ref[1] + 1.0
    cp = cpad_ref[...]
    m = jnp.max(cp, axis=1, keepdims=True)
    e = jnp.exp(cp - m)
    q0 = e / jnp.sum(e, axis=1, keepdims=True)
    xw = xw_ref[...]
    sm = jnp.zeros_like(a)
    for h in range(H):
        qh = q0[0:1, h:h + 1]
        sm = sm + qh * xw[:, h * C:(h + 1) * C]
    y = (a + sm) / deg + b_ref[...]
    out_ref[...] = jnp.where(y > 0, y, jnp.exp(y) - 1.0)


def _epilogue(agg2, xw, deg2, cpad, brow):
    nblk = NPAD // NB
    return pl.pallas_call(
        _epi_body,
        grid=(nblk,),
        in_specs=[pl.BlockSpec((2, NB, C), lambda i: (0, i, 0)),
                  pl.BlockSpec((NB, 5 * C), lambda i: (i, 0)),
                  pl.BlockSpec((2, NB, C), lambda i: (0, i, 0)),
                  pl.BlockSpec((1, C), lambda i: (0, 0)),
                  pl.BlockSpec((1, C), lambda i: (0, 0))],
        out_specs=pl.BlockSpec((NB, C), lambda i: (i, 0)),
        out_shape=jax.ShapeDtypeStruct((NPAD, C), jnp.float32),
    )(agg2, xw, deg2, cpad, brow)


def _head_body(x1_ref, x2_ref, x3_ref, wc1, wc2, wc3, cb_ref,
               w0a, w0b, b0, w1, b1, w2, b2, out_ref, maxs, sums):
    i = pl.program_id(0)
    nblk = pl.num_programs(0)
    dn = (((1,), (1,)), ((), ()))
    y = lax.dot_general(wc1[...], x1_ref[...], dn, preferred_element_type=jnp.float32)
    y = y + lax.dot_general(wc2[...], x2_ref[...], dn, preferred_element_type=jnp.float32)
    y = y + lax.dot_general(wc3[...], x3_ref[...], dn, preferred_element_type=jnp.float32)
    y = y + cb_ref[:, 0:1]
    y = jnp.where(y >= 0, y, 0.2 * y)
    ymax = jnp.broadcast_to(jnp.max(y, axis=1, keepdims=True), (8 * C, C))
    ysum = jnp.broadcast_to(jnp.sum(y, axis=1, keepdims=True), (8 * C, C))

    @pl.when(i == 0)
    def _():
        maxs[...] = ymax
        sums[...] = ysum

    @pl.when(i > 0)
    def _():
        maxs[...] = jnp.maximum(maxs[...], ymax)
        sums[...] = sums[...] + ysum

    @pl.when(i == nblk - 1)
    def _():
        dc = (((0,), (0,)), ((), ()))
        z = lax.dot_general(maxs[...], w0a[...], dc, preferred_element_type=jnp.float32)
        z = z + lax.dot_general(sums[...] * (1.0 / N), w0b[...], dc,
                                preferred_element_type=jnp.float32)
        z = z + b0[...]
        z = jnp.where(z > 0, z, jnp.exp(z) - 1.0)
        z = jnp.dot(z, w1[...], preferred_element_type=jnp.float32) + b1[...]
        z = jnp.where(z > 0, z, jnp.exp(z) - 1.0)
        z = jnp.dot(z, w2[...], preferred_element_type=jnp.float32) + b2[...]
        out_ref[...] = jnp.tanh(z[0:1, :])


def _head(x1, x2, x3, wc1, wc2, wc3, cb8, w0a, w0b, b0, w1, b1, w2p, b2p):
    nblk = N // HB
    return pl.pallas_call(
        _head_body,
        grid=(nblk,),
        in_specs=[pl.BlockSpec((HB, C), lambda i: (i, 0)),
                  pl.BlockSpec((HB, C), lambda i: (i, 0)),
                  pl.BlockSpec((HB, C), lambda i: (i, 0)),
                  pl.BlockSpec((8 * C, C), lambda i: (0, 0)),
                  pl.BlockSpec((8 * C, C), lambda i: (0, 0)),
                  pl.BlockSpec((8 * C, C), lambda i: (0, 0)),
                  pl.BlockSpec((8 * C, 8), lambda i: (0, 0)),
                  pl.BlockSpec((8 * C, 512), lambda i: (0, 0)),
                  pl.BlockSpec((8 * C, 512), lambda i: (0, 0)),
                  pl.BlockSpec((1, 512), lambda i: (0, 0)),
                  pl.BlockSpec((512, 256), lambda i: (0, 0)),
                  pl.BlockSpec((1, 256), lambda i: (0, 0)),
                  pl.BlockSpec((256, C), lambda i: (0, 0)),
                  pl.BlockSpec((1, C), lambda i: (0, 0))],
        out_specs=pl.BlockSpec((1, C), lambda i: (0, 0)),
        out_shape=jax.ShapeDtypeStruct((1, C), jnp.float32),
        scratch_shapes=[pltpu.VMEM((8 * C, C), jnp.float32),
                        pltpu.VMEM((8 * C, C), jnp.float32)],
    )(x1, x2, x3, wc1, wc2, wc3, cb8, w0a, w0b, b0, w1, b1, w2p, b2p)


EP = 163840
EPT = EP // 32
CH = 32
AGGR = 10112
AGGROW = AGGR // 16
NROW = NPAD // 16


def _lane_gather(v, idx):
    dn = lax.GatherDimensionNumbers(offset_dims=(), collapsed_slice_dims=(0,),
                                    start_index_map=(0,))
    return lax.gather(v, idx[:, None], dn, (1,),
                      mode=lax.GatherScatterMode.PROMISE_IN_BOUNDS)


def _lane_bcast(v, t):
    return _lane_gather(v, jnp.full((16,), t, jnp.int32))


def _edge_body(xw4, xu, sdr, dstr, cvec, zer, out,
               agg_sh, cvec_v, ixsd0, ixsd1, ixd0, ixd1, ixd2, ixd3,
               ix40, ix41, rows0, rows1, uu_v, msg_v,
               semr0, semr1, semu, semw, semi):
    cid = lax.axis_index("c")
    sid = lax.axis_index("s")
    wid = sid * 2 + cid
    pltpu.sync_copy(zer, agg_sh.at[pl.ds(sid * AGGROW, AGGROW)])
    pltpu.sync_copy(cvec, cvec_v)
    plsc.subcore_barrier()

    lane = lax.iota(jnp.int32, 16)
    rot1 = jnp.bitwise_or(jnp.bitwise_and(lane, 12),
                          jnp.bitwise_and(lane + 1, 3))
    rot2 = jnp.bitwise_or(jnp.bitwise_and(lane, 12),
                          jnp.bitwise_and(lane + 2, 3))
    ixsd = (ixsd0, ixsd1)
    ixd = (ixd0, ixd1, ixd2, ixd3)
    ix4 = (ix40, ix41)
    rows = (rows0, rows1)
    semr = (semr0, semr1)
    NCH = EPT // CH
    srow = wid * NCH

    def fire_idx(g, s2, s4):
        pltpu.async_copy(sdr.at[g], ixsd[s2], semi)
        pltpu.async_copy(dstr.at[g], ixd[s4], semi)

    def wait_idx(s2, s4):
        pltpu.make_async_copy(sdr.at[0], ixsd[s2], semi).wait()
        pltpu.make_async_copy(dstr.at[0], ixd[s4], semi).wait()

    def build_ix4(s2):
        for j in range(CH // 16):
            s16 = ixsd[s2][pl.ds(j * 16, 16)]
            s4v = s16 * 4
            for h in range(H):
                ix4[s2][h, pl.ds(j * 16, 16)] = s4v + h

    def fire_rows(s2, b):
        for h in range(H):
            pltpu.async_copy(xw4.at[ix4[s2].at[h]], rows[b].at[h], semr[b])

    def wait_rows(s2, b):
        for h in range(H):
            pltpu.make_async_copy(xw4.at[ix4[s2].at[h]], rows[b].at[h],
                                  semr[b]).wait()

    fire_idx(srow, 0, 0)
    wait_idx(0, 0)
    build_ix4(0)
    fire_rows(0, 0)
    pltpu.async_copy(xu.at[ixsd0], uu_v, semu)
    fire_idx(srow + 1, 1, 1)
    fire_idx(srow + 2, 0, 2)

    def quad(Q, carry):
        for r in range(4):
            g = Q * 4 + r
            b = r & 1
            rv = rows[b]
            rn = (r + 1) & 3
            if r > 0:
                pltpu.make_async_copy(
                    msg_v, agg_sh.at[ixd[r - 1]], semw).wait()
            else:
                @pl.when(Q > 0)
                def _():
                    pltpu.make_async_copy(
                        msg_v, agg_sh.at[ixd[3]], semw).wait()

            @pl.when(g + 1 < NCH)
            def _():
                wait_idx((r + 1) & 1, rn)
                build_ix4((r + 1) & 1)
                fire_rows((r + 1) & 1, 1 - b)

            @pl.when(g + 3 < NCH)
            def _():
                fire_idx(srow + g + 3, (r + 3) & 1, (r + 3) & 3)

            wait_rows(b, b)
            pltpu.make_async_copy(xu.at[ixsd[b]], uu_v, semu).wait()
            cv = cvec_v[...]

            def edge(e, c3):
                l = uu_v[e, pl.ds(0, 16)] - uu_v[CH + e, pl.ds(0, 16)] + cv
                m = jnp.maximum(l, _lane_gather(l, rot1))
                m = jnp.maximum(m, _lane_gather(m, rot2))
                ex = jnp.exp(l - m)
                sm = ex + _lane_gather(ex, rot1)
                sm = sm + _lane_gather(sm, rot2)
                q = ex / sm
                qs = [_lane_bcast(q, h) for h in range(H)]
                for k in range(C // 16):
                    acc = qs[0] * rv[0, e, pl.ds(k * 16, 16)]
                    acc = acc + qs[1] * rv[1, e, pl.ds(k * 16, 16)]
                    acc = acc + qs[2] * rv[2, e, pl.ds(k * 16, 16)]
                    acc = acc + qs[3] * rv[3, e, pl.ds(k * 16, 16)]
                    msg_v[e, pl.ds(k * 16, 16)] = acc
                return c3

            lax.fori_loop(0, CH, edge, 0, unroll=4)

            @pl.when(g + 1 < NCH)
            def _():
                pltpu.async_copy(xu.at[ixsd[(r + 1) & 1]], uu_v, semu)

            pltpu.async_copy(msg_v, agg_sh.at[ixd[r]], semw, add=True)
        return carry

    lax.fori_loop(0, NCH // 4, quad, 0)
    pltpu.make_async_copy(msg_v, agg_sh.at[ixd[3]], semw).wait()
    plsc.subcore_barrier()
    pltpu.sync_copy(agg_sh.at[pl.ds(sid * AGGROW, AGGROW)],
                    out.at[cid, pl.ds(sid * AGGROW, AGGROW)])


@functools.partial(
    pl.kernel,
    out_type=jax.ShapeDtypeStruct((2, NPAD, C), jnp.float32),
    mesh=plsc.VectorSubcoreMesh(core_axis_name="c", subcore_axis_name="s"),
    compiler_params=pltpu.CompilerParams(needs_layout_passes=False),
    scratch_types=[
        pltpu.VMEM_SHARED((AGGR, C), jnp.float32),
        pltpu.VMEM((16,), jnp.float32),
        pltpu.VMEM((2 * CH,), jnp.int32),
        pltpu.VMEM((2 * CH,), jnp.int32),
        pltpu.VMEM((CH,), jnp.int32),
        pltpu.VMEM((CH,), jnp.int32),
        pltpu.VMEM((CH,), jnp.int32),
        pltpu.VMEM((CH,), jnp.int32),
        pltpu.VMEM((H, CH), jnp.int32),
        pltpu.VMEM((H, CH), jnp.int32),
        pltpu.VMEM((H, CH, C), jnp.float32),
        pltpu.VMEM((H, CH, C), jnp.float32),
        pltpu.VMEM((2 * CH, C), jnp.float32),
        pltpu.VMEM((CH, C), jnp.float32),
        pltpu.SemaphoreType.DMA,
        pltpu.SemaphoreType.DMA,
        pltpu.SemaphoreType.DMA,
        pltpu.SemaphoreType.DMA,
        pltpu.SemaphoreType.DMA,
    ],
)
def _sc_edge(xw4, xu, sdr, dstr, cvec, zer, out, *rest):
    _edge_body(xw4, xu, sdr, dstr, cvec, zer, out, *rest)


def _deg_body(dstr, onesr, zer, out, deg_sh, idx_d, ones_v, sem):
    cid = lax.axis_index("c")
    sid = lax.axis_index("s")
    wid = sid * 2 + cid
    pltpu.sync_copy(zer, deg_sh.at[pl.ds(sid * AGGROW, AGGROW)])
    pltpu.sync_copy(onesr, ones_v)
    plsc.subcore_barrier()

    def chunk(g, carry):
        base = wid * (EPT // CH) + g
        pltpu.sync_copy(dstr.at[base], idx_d)
        pltpu.sync_copy(ones_v, deg_sh.at[idx_d], add=True)
        return carry

    lax.fori_loop(0, EPT // CH, chunk, 0)
    plsc.subcore_barrier()
    pltpu.sync_copy(deg_sh.at[pl.ds(sid * AGGROW, AGGROW)],
                    out.at[cid, pl.ds(sid * AGGROW, AGGROW)])


@functools.partial(
    pl.kernel,
    out_type=jax.ShapeDtypeStruct((2, NPAD, C), jnp.float32),
    mesh=plsc.VectorSubcoreMesh(core_axis_name="c", subcore_axis_name="s"),
    compiler_params=pltpu.CompilerParams(needs_layout_passes=False),
    scratch_types=[
        pltpu.VMEM_SHARED((AGGR, C), jnp.float32),
        pltpu.VMEM((CH,), jnp.int32),
        pltpu.VMEM((CH, C), jnp.float32),
        pltpu.SemaphoreType.DMA,
    ],
)
def _sc_deg(dstr, onesr, zer, out, *rest):
    _deg_body(dstr, onesr, zer, out, *rest)


def kernel(verts, params, edges):
    src, dst = edges[0], edges[1]
    E = src.shape[0]
    x = jnp.pad(verts, ((0, NPAD - N), (0, 0)))

    srcp = jnp.concatenate([src, jnp.zeros((EP - E,), src.dtype)]).reshape(
        EP // CH, CH)
    dstp = jnp.concatenate([dst, jnp.full((EP - E,), N, dst.dtype)]).reshape(
        EP // CH, CH)
    sdp = jnp.concatenate([srcp, dstp], axis=1)

    zer128 = jnp.zeros((AGGROW, C), jnp.float32)
    ones128 = jnp.ones((CH, C), jnp.float32)

    deg2 = _sc_deg(dstp, ones128, zer128)

    xs = []
    for p in params['convs']:
        U128 = jnp.concatenate(
            [jnp.tile(p['U'], (1, 4)), jnp.zeros((C, C - 16), jnp.float32)],
            axis=1)
        cpad = jnp.full((1, C), NEG, jnp.float32).at[0, :H].set(p['c'])
        cvec = jnp.tile(p['c'], 4)
        brow = p['b'].reshape(1, C)
        xw, xu = _mm(x, p['W'], U128)
        xw4 = xw.reshape(NPAD * 4, C)
        agg2 = _sc_edge(xw4, xu, sdp, dstp, cvec, zer128)
        x = _epilogue(agg2, xw, deg2, cpad, brow)
        xs.append(x[:N])

    wc = params['conv1d_w']
    cb8 = jnp.broadcast_to(params['conv1d_b'][:, None], (8 * C, 8))
    (W0, b0), (W1, b1), (W2, b2) = params['lins']
    w2p = jnp.pad(W2, ((0, 0), (0, C - W2.shape[1])))
    b2p = jnp.pad(b2, (0, C - b2.shape[0])).reshape(1, C)
    out = _head(xs[0], xs[1], xs[2],
                wc[:, 0:C], wc[:, C:2 * C], wc[:, 2 * C:3 * C], cb8,
                W0[:8 * C], W0[8 * C:], b0.reshape(1, 512),
                W1, b1.reshape(1, 256), w2p, b2p)
    return out[:, :10]


# speedup vs baseline: 1.0062x; 1.0062x over previous
import functools
import jax
import jax.numpy as jnp
from jax import lax
from jax.experimental import pallas as pl
from jax.experimental.pallas import tpu as pltpu
from jax.experimental.pallas import tpu_sc as plsc

H = 4
N = 10000
C = 128
NPAD = 10112
NB = 128
HB = 400
NEG = -1e30


def _mm_body(x_ref, w_ref, u_ref, xw_ref, xu_ref):
    x = x_ref[...]
    xw_ref[...] = jnp.dot(x, w_ref[...], preferred_element_type=jnp.float32)
    xu_ref[...] = jnp.dot(x, u_ref[...], preferred_element_type=jnp.float32)


def _mm(x, W, U128):
    nblk = NPAD // NB
    return pl.pallas_call(
        _mm_body,
        grid=(nblk,),
        in_specs=[pl.BlockSpec((NB, C), lambda i: (i, 0)),
                  pl.BlockSpec((C, 4 * C), lambda i: (0, 0)),
                  pl.BlockSpec((C, C), lambda i: (0, 0))],
        out_specs=[pl.BlockSpec((NB, 4 * C), lambda i: (i, 0)),
                   pl.BlockSpec((NB, C), lambda i: (i, 0))],
        out_shape=[jax.ShapeDtypeStruct((NPAD, 4 * C), jnp.float32),
                   jax.ShapeDtypeStruct((NPAD, C), jnp.float32)],
    )(x, W, U128)


def _epi_body(agg_ref, xw_ref, deg_ref, cpad_ref, b_ref, out_ref):
    a = agg_ref[0] + agg_ref[1]
    deg = deg_ref[0] + deg_ref[1] + 1.0
    cp = cpad_ref[...]
    m = jnp.max(cp, axis=1, keepdims=True)
    e = jnp.exp(cp - m)
    q0 = e / jnp.sum(e, axis=1, keepdims=True)
    xw = xw_ref[...]
    sm = jnp.zeros_like(a)
    for h in range(H):
        qh = q0[0:1, h:h + 1]
        sm = sm + qh * xw[:, h * C:(h + 1) * C]
    y = (a + sm) / deg + b_ref[...]
    out_ref[...] = jnp.where(y > 0, y, jnp.exp(y) - 1.0)


def _epilogue(agg2, xw, deg2, cpad, brow):
    nblk = NPAD // NB
    return pl.pallas_call(
        _epi_body,
        grid=(nblk,),
        in_specs=[pl.BlockSpec((2, NB, C), lambda i: (0, i, 0)),
                  pl.BlockSpec((NB, 5 * C), lambda i: (i, 0)),
                  pl.BlockSpec((2, NB, C), lambda i: (0, i, 0)),
                  pl.BlockSpec((1, C), lambda i: (0, 0)),
                  pl.BlockSpec((1, C), lambda i: (0, 0))],
        out_specs=pl.BlockSpec((NB, C), lambda i: (i, 0)),
        out_shape=jax.ShapeDtypeStruct((NPAD, C), jnp.float32),
    )(agg2, xw, deg2, cpad, brow)


def _head_body(x1_ref, x2_ref, x3_ref, wc1, wc2, wc3, cb_ref,
               w0a, w0b, b0, w1, b1, w2, b2, out_ref, maxs, sums):
    i = pl.program_id(0)
    nblk = pl.num_programs(0)
    dn = (((1,), (1,)), ((), ()))
    y = lax.dot_general(wc1[...], x1_ref[...], dn, preferred_element_type=jnp.float32)
    y = y + lax.dot_general(wc2[...], x2_ref[...], dn, preferred_element_type=jnp.float32)
    y = y + lax.dot_general(wc3[...], x3_ref[...], dn, preferred_element_type=jnp.float32)
    y = y + cb_ref[:, 0:1]
    y = jnp.where(y >= 0, y, 0.2 * y)
    ymax = jnp.broadcast_to(jnp.max(y, axis=1, keepdims=True), (8 * C, C))
    ysum = jnp.broadcast_to(jnp.sum(y, axis=1, keepdims=True), (8 * C, C))

    @pl.when(i == 0)
    def _():
        maxs[...] = ymax
        sums[...] = ysum

    @pl.when(i > 0)
    def _():
        maxs[...] = jnp.maximum(maxs[...], ymax)
        sums[...] = sums[...] + ysum

    @pl.when(i == nblk - 1)
    def _():
        dc = (((0,), (0,)), ((), ()))
        z = lax.dot_general(maxs[...], w0a[...], dc, preferred_element_type=jnp.float32)
        z = z + lax.dot_general(sums[...] * (1.0 / N), w0b[...], dc,
                                preferred_element_type=jnp.float32)
        z = z + b0[...]
        z = jnp.where(z > 0, z, jnp.exp(z) - 1.0)
        z = jnp.dot(z, w1[...], preferred_element_type=jnp.float32) + b1[...]
        z = jnp.where(z > 0, z, jnp.exp(z) - 1.0)
        z = jnp.dot(z, w2[...], preferred_element_type=jnp.float32) + b2[...]
        out_ref[...] = jnp.tanh(z[0:1, :])


def _head(x1, x2, x3, wc1, wc2, wc3, cb8, w0a, w0b, b0, w1, b1, w2p, b2p):
    nblk = N // HB
    return pl.pallas_call(
        _head_body,
        grid=(nblk,),
        in_specs=[pl.BlockSpec((HB, C), lambda i: (i, 0)),
                  pl.BlockSpec((HB, C), lambda i: (i, 0)),
                  pl.BlockSpec((HB, C), lambda i: (i, 0)),
                  pl.BlockSpec((8 * C, C), lambda i: (0, 0)),
                  pl.BlockSpec((8 * C, C), lambda i: (0, 0)),
                  pl.BlockSpec((8 * C, C), lambda i: (0, 0)),
                  pl.BlockSpec((8 * C, 8), lambda i: (0, 0)),
                  pl.BlockSpec((8 * C, 512), lambda i: (0, 0)),
                  pl.BlockSpec((8 * C, 512), lambda i: (0, 0)),
                  pl.BlockSpec((1, 512), lambda i: (0, 0)),
                  pl.BlockSpec((512, 256), lambda i: (0, 0)),
                  pl.BlockSpec((1, 256), lambda i: (0, 0)),
                  pl.BlockSpec((256, C), lambda i: (0, 0)),
                  pl.BlockSpec((1, C), lambda i: (0, 0))],
        out_specs=pl.BlockSpec((1, C), lambda i: (0, 0)),
        out_shape=jax.ShapeDtypeStruct((1, C), jnp.float32),
        scratch_shapes=[pltpu.VMEM((8 * C, C), jnp.float32),
                        pltpu.VMEM((8 * C, C), jnp.float32)],
    )(x1, x2, x3, wc1, wc2, wc3, cb8, w0a, w0b, b0, w1, b1, w2p, b2p)


EP = 163840
EPT = EP // 32
CH = 32
AGGR = 10112
AGGROW = AGGR // 16
NROW = NPAD // 16


def _lane_gather(v, idx):
    dn = lax.GatherDimensionNumbers(offset_dims=(), collapsed_slice_dims=(0,),
                                    start_index_map=(0,))
    return lax.gather(v, idx[:, None], dn, (1,),
                      mode=lax.GatherScatterMode.PROMISE_IN_BOUNDS)


def _lane_bcast(v, t):
    return _lane_gather(v, jnp.full((16,), t, jnp.int32))


def _edge_body(xw4, xu, idxr, dstr, cvec, zer, out,
               agg_sh, cvec_v, ix0, ix1, ixd0, ixd1, ixd2, ixd3,
               rows0, rows1, uu_v, msg_v,
               semr0, semr1, semu, semw, semi):
    cid = lax.axis_index("c")
    sid = lax.axis_index("s")
    wid = sid * 2 + cid
    pltpu.sync_copy(zer, agg_sh.at[pl.ds(sid * AGGROW, AGGROW)])
    pltpu.sync_copy(cvec, cvec_v)
    plsc.subcore_barrier()

    lane = lax.iota(jnp.int32, 16)
    rot1 = jnp.bitwise_or(jnp.bitwise_and(lane, 12),
                          jnp.bitwise_and(lane + 1, 3))
    rot2 = jnp.bitwise_or(jnp.bitwise_and(lane, 12),
                          jnp.bitwise_and(lane + 2, 3))
    ix = (ix0, ix1)
    ixd = (ixd0, ixd1, ixd2, ixd3)
    rows = (rows0, rows1)
    semr = (semr0, semr1)
    NCH = EPT // CH
    srow = wid * NCH

    def fire_idx(g, s2, s4):
        pltpu.async_copy(idxr.at[g], ix[s2], semi)
        pltpu.async_copy(dstr.at[g], ixd[s4], semi)

    def wait_idx(s2, s4):
        pltpu.make_async_copy(idxr.at[0], ix[s2], semi).wait()
        pltpu.make_async_copy(dstr.at[0], ixd[s4], semi).wait()

    def fire_rows(s2, b):
        for h in range(H):
            pltpu.async_copy(xw4.at[ix[s2].at[pl.ds(h * CH, CH)]],
                             rows[b].at[h], semr[b])

    def wait_rows(s2, b):
        for h in range(H):
            pltpu.make_async_copy(xw4.at[ix[s2].at[pl.ds(h * CH, CH)]],
                                  rows[b].at[h], semr[b]).wait()

    def fire_uu(s2):
        pltpu.async_copy(xu.at[ix[s2].at[pl.ds(4 * CH, 2 * CH)]], uu_v, semu)

    def wait_uu(s2):
        pltpu.make_async_copy(xu.at[ix[s2].at[pl.ds(4 * CH, 2 * CH)]], uu_v,
                              semu).wait()

    fire_idx(srow, 0, 0)
    wait_idx(0, 0)
    fire_rows(0, 0)
    fire_uu(0)
    fire_idx(srow + 1, 1, 1)
    fire_idx(srow + 2, 0, 2)

    def quad(Q, carry):
        for r in range(4):
            g = Q * 4 + r
            b = r & 1
            rv = rows[b]
            rn = (r + 1) & 3
            if r > 0:
                pltpu.make_async_copy(
                    msg_v, agg_sh.at[ixd[r - 1]], semw).wait()
            else:
                @pl.when(Q > 0)
                def _():
                    pltpu.make_async_copy(
                        msg_v, agg_sh.at[ixd[3]], semw).wait()

            @pl.when(g + 1 < NCH)
            def _():
                wait_idx((r + 1) & 1, rn)
                fire_rows((r + 1) & 1, 1 - b)

            @pl.when(g + 3 < NCH)
            def _():
                fire_idx(srow + g + 3, (r + 3) & 1, (r + 3) & 3)

            wait_rows(b, b)
            wait_uu(b)
            cv = cvec_v[...]

            def edge(e, c3):
                l = uu_v[e, pl.ds(0, 16)] - uu_v[CH + e, pl.ds(0, 16)] + cv
                m = jnp.maximum(l, _lane_gather(l, rot1))
                m = jnp.maximum(m, _lane_gather(m, rot2))
                ex = jnp.exp(l - m)
                sm = ex + _lane_gather(ex, rot1)
                sm = sm + _lane_gather(sm, rot2)
                q = ex / sm
                qs = [_lane_bcast(q, h) for h in range(H)]
                for k in range(C // 16):
                    acc = qs[0] * rv[0, e, pl.ds(k * 16, 16)]
                    acc = acc + qs[1] * rv[1, e, pl.ds(k * 16, 16)]
                    acc = acc + qs[2] * rv[2, e, pl.ds(k * 16, 16)]
                    acc = acc + qs[3] * rv[3, e, pl.ds(k * 16, 16)]
                    msg_v[e, pl.ds(k * 16, 16)] = acc
                return c3

            lax.fori_loop(0, CH, edge, 0, unroll=4)

            @pl.when(g + 1 < NCH)
            def _():
                fire_uu((r + 1) & 1)

            pltpu.async_copy(msg_v, agg_sh.at[ixd[r]], semw, add=True)
        return carry

    lax.fori_loop(0, NCH // 4, quad, 0)
    pltpu.make_async_copy(msg_v, agg_sh.at[ixd[3]], semw).wait()
    plsc.subcore_barrier()
    pltpu.sync_copy(agg_sh.at[pl.ds(sid * AGGROW, AGGROW)],
                    out.at[cid, pl.ds(sid * AGGROW, AGGROW)])


@functools.partial(
    pl.kernel,
    out_type=jax.ShapeDtypeStruct((2, NPAD, C), jnp.float32),
    mesh=plsc.VectorSubcoreMesh(core_axis_name="c", subcore_axis_name="s"),
    compiler_params=pltpu.CompilerParams(needs_layout_passes=False),
    scratch_types=[
        pltpu.VMEM_SHARED((AGGR, C), jnp.float32),
        pltpu.VMEM((16,), jnp.float32),
        pltpu.VMEM((6 * CH,), jnp.int32),
        pltpu.VMEM((6 * CH,), jnp.int32),
        pltpu.VMEM((CH,), jnp.int32),
        pltpu.VMEM((CH,), jnp.int32),
        pltpu.VMEM((CH,), jnp.int32),
        pltpu.VMEM((CH,), jnp.int32),
        pltpu.VMEM((H, CH, C), jnp.float32),
        pltpu.VMEM((H, CH, C), jnp.float32),
        pltpu.VMEM((2 * CH, C), jnp.float32),
        pltpu.VMEM((CH, C), jnp.float32),
        pltpu.SemaphoreType.DMA,
        pltpu.SemaphoreType.DMA,
        pltpu.SemaphoreType.DMA,
        pltpu.SemaphoreType.DMA,
        pltpu.SemaphoreType.DMA,
    ],
)
def _sc_edge(xw4, xu, idxr, dstr, cvec, zer, out, *rest):
    _edge_body(xw4, xu, idxr, dstr, cvec, zer, out, *rest)


def _deg_body(dstr, onesr, zer, out, deg_sh, idx_d, ones_v, sem):
    cid = lax.axis_index("c")
    sid = lax.axis_index("s")
    wid = sid * 2 + cid
    pltpu.sync_copy(zer, deg_sh.at[pl.ds(sid * AGGROW, AGGROW)])
    pltpu.sync_copy(onesr, ones_v)
    plsc.subcore_barrier()

    def chunk(g, carry):
        base = wid * (EPT // CH) + g
        pltpu.sync_copy(dstr.at[base], idx_d)
        pltpu.sync_copy(ones_v, deg_sh.at[idx_d], add=True)
        return carry

    lax.fori_loop(0, EPT // CH, chunk, 0)
    plsc.subcore_barrier()
    pltpu.sync_copy(deg_sh.at[pl.ds(sid * AGGROW, AGGROW)],
                    out.at[cid, pl.ds(sid * AGGROW, AGGROW)])


@functools.partial(
    pl.kernel,
    out_type=jax.ShapeDtypeStruct((2, NPAD, C), jnp.float32),
    mesh=plsc.VectorSubcoreMesh(core_axis_name="c", subcore_axis_name="s"),
    compiler_params=pltpu.CompilerParams(needs_layout_passes=False),
    scratch_types=[
        pltpu.VMEM_SHARED((AGGR, C), jnp.float32),
        pltpu.VMEM((CH,), jnp.int32),
        pltpu.VMEM((CH, C), jnp.float32),
        pltpu.SemaphoreType.DMA,
    ],
)
def _sc_deg(dstr, onesr, zer, out, *rest):
    _deg_body(dstr, onesr, zer, out, *rest)


def kernel(verts, params, edges):
    src, dst = edges[0], edges[1]
    E = src.shape[0]
    x = jnp.pad(verts, ((0, NPAD - N), (0, 0)))

    srcp = jnp.concatenate([src, jnp.zeros((EP - E,), src.dtype)]).reshape(
        EP // CH, CH)
    dstp = jnp.concatenate([dst, jnp.full((EP - E,), N, dst.dtype)]).reshape(
        EP // CH, CH)
    s4 = srcp * 4
    idxc = jnp.stack([s4, s4 + 1, s4 + 2, s4 + 3, srcp, dstp],
                     axis=1).reshape(EP // CH, 6 * CH)

    zer128 = jnp.zeros((AGGROW, C), jnp.float32)
    ones128 = jnp.ones((CH, C), jnp.float32)

    deg2 = _sc_deg(dstp, ones128, zer128)

    xs = []
    for p in params['convs']:
        U128 = jnp.concatenate(
            [jnp.tile(p['U'], (1, 4)), jnp.zeros((C, C - 16), jnp.float32)],
            axis=1)
        cpad = jnp.full((1, C), NEG, jnp.float32).at[0, :H].set(p['c'])
        cvec = jnp.tile(p['c'], 4)
        brow = p['b'].reshape(1, C)
        xw, xu = _mm(x, p['W'], U128)
        xw4 = xw.reshape(NPAD * 4, C)
        agg2 = _sc_edge(xw4, xu, idxc, dstp, cvec, zer128)
        x = _epilogue(agg2, xw, deg2, cpad, brow)
        xs.append(x[:N])

    wc = params['conv1d_w']
    cb8 = jnp.broadcast_to(params['conv1d_b'][:, None], (8 * C, 8))
    (W0, b0), (W1, b1), (W2, b2) = params['lins']
    w2p = jnp.pad(W2, ((0, 0), (0, C - W2.shape[1])))
    b2p = jnp.pad(b2, (0, C - b2.shape[0])).reshape(1, C)
    out = _head(xs[0], xs[1], xs[2],
                wc[:, 0:C], wc[:, C:2 * C], wc[:, 2 * C:3 * C], cb8,
                W0[:8 * C], W0[8 * C:], b0.reshape(1, 512),
                W1, b1.reshape(1, 256), w2p, b2p)
    return out[:, :10]


# speedup vs baseline: 1.0216x; 1.0154x over previous
import functools
import jax
import jax.numpy as jnp
from jax import lax
from jax.experimental import pallas as pl
from jax.experimental.pallas import tpu as pltpu
from jax.experimental.pallas import tpu_sc as plsc

H = 4
N = 10000
C = 128
NPAD = 10112
NB = 128
HB = 400
NEG = -1e30


def _mm_body(x_ref, w_ref, u_ref, xw_ref, xu_ref):
    x = x_ref[...]
    xw_ref[...] = jnp.dot(x, w_ref[...], preferred_element_type=jnp.float32)
    xu_ref[...] = jnp.dot(x, u_ref[...], preferred_element_type=jnp.float32)


def _mm(x, W, U128):
    nblk = NPAD // NB
    return pl.pallas_call(
        _mm_body,
        grid=(nblk,),
        in_specs=[pl.BlockSpec((NB, C), lambda i: (i, 0)),
                  pl.BlockSpec((C, 4 * C), lambda i: (0, 0)),
                  pl.BlockSpec((C, C), lambda i: (0, 0))],
        out_specs=[pl.BlockSpec((NB, 4 * C), lambda i: (i, 0)),
                   pl.BlockSpec((NB, C), lambda i: (i, 0))],
        out_shape=[jax.ShapeDtypeStruct((NPAD, 4 * C), jnp.float32),
                   jax.ShapeDtypeStruct((NPAD, C), jnp.float32)],
    )(x, W, U128)


def _epi_body(agg_ref, xw_ref, deg_ref, cpad_ref, b_ref, out_ref):
    a = agg_ref[0] + agg_ref[1]
    deg = deg_ref[0] + deg_ref[1] + 1.0
    cp = cpad_ref[...]
    m = jnp.max(cp, axis=1, keepdims=True)
    e = jnp.exp(cp - m)
    q0 = e / jnp.sum(e, axis=1, keepdims=True)
    xw = xw_ref[...]
    sm = jnp.zeros_like(a)
    for h in range(H):
        qh = q0[0:1, h:h + 1]
        sm = sm + qh * xw[:, h * C:(h + 1) * C]
    y = (a + sm) / deg + b_ref[...]
    out_ref[...] = jnp.where(y > 0, y, jnp.exp(y) - 1.0)


def _epilogue(agg2, xw, deg2, cpad, brow):
    nblk = NPAD // NB
    return pl.pallas_call(
        _epi_body,
        grid=(nblk,),
        in_specs=[pl.BlockSpec((2, NB, C), lambda i: (0, i, 0)),
                  pl.BlockSpec((NB, 5 * C), lambda i: (i, 0)),
                  pl.BlockSpec((2, NB, C), lambda i: (0, i, 0)),
                  pl.BlockSpec((1, C), lambda i: (0, 0)),
                  pl.BlockSpec((1, C), lambda i: (0, 0))],
        out_specs=pl.BlockSpec((NB, C), lambda i: (i, 0)),
        out_shape=jax.ShapeDtypeStruct((NPAD, C), jnp.float32),
    )(agg2, xw, deg2, cpad, brow)


def _head_body(x1_ref, x2_ref, x3_ref, wc1, wc2, wc3, cb_ref,
               w0a, w0b, b0, w1, b1, w2, b2, out_ref, maxs, sums):
    i = pl.program_id(0)
    nblk = pl.num_programs(0)
    dn = (((1,), (1,)), ((), ()))
    y = lax.dot_general(wc1[...], x1_ref[...], dn, preferred_element_type=jnp.float32)
    y = y + lax.dot_general(wc2[...], x2_ref[...], dn, preferred_element_type=jnp.float32)
    y = y + lax.dot_general(wc3[...], x3_ref[...], dn, preferred_element_type=jnp.float32)
    y = y + cb_ref[:, 0:1]
    y = jnp.where(y >= 0, y, 0.2 * y)
    ymax = jnp.broadcast_to(jnp.max(y, axis=1, keepdims=True), (8 * C, C))
    ysum = jnp.broadcast_to(jnp.sum(y, axis=1, keepdims=True), (8 * C, C))

    @pl.when(i == 0)
    def _():
        maxs[...] = ymax
        sums[...] = ysum

    @pl.when(i > 0)
    def _():
        maxs[...] = jnp.maximum(maxs[...], ymax)
        sums[...] = sums[...] + ysum

    @pl.when(i == nblk - 1)
    def _():
        dc = (((0,), (0,)), ((), ()))
        z = lax.dot_general(maxs[...], w0a[...], dc, preferred_element_type=jnp.float32)
        z = z + lax.dot_general(sums[...] * (1.0 / N), w0b[...], dc,
                                preferred_element_type=jnp.float32)
        z = z + b0[...]
        z = jnp.where(z > 0, z, jnp.exp(z) - 1.0)
        z = jnp.dot(z, w1[...], preferred_element_type=jnp.float32) + b1[...]
        z = jnp.where(z > 0, z, jnp.exp(z) - 1.0)
        z = jnp.dot(z, w2[...], preferred_element_type=jnp.float32) + b2[...]
        out_ref[...] = jnp.tanh(z[0:1, :])


def _head(x1, x2, x3, wc1, wc2, wc3, cb8, w0a, w0b, b0, w1, b1, w2p, b2p):
    nblk = N // HB
    return pl.pallas_call(
        _head_body,
        grid=(nblk,),
        in_specs=[pl.BlockSpec((HB, C), lambda i: (i, 0)),
                  pl.BlockSpec((HB, C), lambda i: (i, 0)),
                  pl.BlockSpec((HB, C), lambda i: (i, 0)),
                  pl.BlockSpec((8 * C, C), lambda i: (0, 0)),
                  pl.BlockSpec((8 * C, C), lambda i: (0, 0)),
                  pl.BlockSpec((8 * C, C), lambda i: (0, 0)),
                  pl.BlockSpec((8 * C, 8), lambda i: (0, 0)),
                  pl.BlockSpec((8 * C, 512), lambda i: (0, 0)),
                  pl.BlockSpec((8 * C, 512), lambda i: (0, 0)),
                  pl.BlockSpec((1, 512), lambda i: (0, 0)),
                  pl.BlockSpec((512, 256), lambda i: (0, 0)),
                  pl.BlockSpec((1, 256), lambda i: (0, 0)),
                  pl.BlockSpec((256, C), lambda i: (0, 0)),
                  pl.BlockSpec((1, C), lambda i: (0, 0))],
        out_specs=pl.BlockSpec((1, C), lambda i: (0, 0)),
        out_shape=jax.ShapeDtypeStruct((1, C), jnp.float32),
        scratch_shapes=[pltpu.VMEM((8 * C, C), jnp.float32),
                        pltpu.VMEM((8 * C, C), jnp.float32)],
    )(x1, x2, x3, wc1, wc2, wc3, cb8, w0a, w0b, b0, w1, b1, w2p, b2p)


EP = 163840
EPT = EP // 32
CH = 32
AGGR = 10112
AGGROW = AGGR // 16
NROW = NPAD // 16


def _lane_gather(v, idx):
    dn = lax.GatherDimensionNumbers(offset_dims=(), collapsed_slice_dims=(0,),
                                    start_index_map=(0,))
    return lax.gather(v, idx[:, None], dn, (1,),
                      mode=lax.GatherScatterMode.PROMISE_IN_BOUNDS)


def _lane_bcast(v, t):
    return _lane_gather(v, jnp.full((16,), t, jnp.int32))


def _edge_body(xw4, xu, idx4r, sdr, dstr, cvec, zer, out,
               agg_sh, cvec_v,
               ixh0a, ixh0b, ixh0c, ixh0d, ixh1a, ixh1b, ixh1c, ixh1d,
               ixh2a, ixh2b, ixh2c, ixh2d, ixh3a, ixh3b, ixh3c, ixh3d,
               ixu0, ixu1, ixu2, ixu3, ixd0, ixd1, ixd2, ixd3,
               rows0, rows1, uu_v, msg_v,
               semr0, semr1, semu, semw, semi):
    cid = lax.axis_index("c")
    sid = lax.axis_index("s")
    wid = sid * 2 + cid
    pltpu.sync_copy(zer, agg_sh.at[pl.ds(sid * AGGROW, AGGROW)])
    pltpu.sync_copy(cvec, cvec_v)
    plsc.subcore_barrier()

    lane = lax.iota(jnp.int32, 16)
    rot1 = jnp.bitwise_or(jnp.bitwise_and(lane, 12),
                          jnp.bitwise_and(lane + 1, 3))
    rot2 = jnp.bitwise_or(jnp.bitwise_and(lane, 12),
                          jnp.bitwise_and(lane + 2, 3))
    ixh = ((ixh0a, ixh0b, ixh0c, ixh0d), (ixh1a, ixh1b, ixh1c, ixh1d),
           (ixh2a, ixh2b, ixh2c, ixh2d), (ixh3a, ixh3b, ixh3c, ixh3d))
    ixu = (ixu0, ixu1, ixu2, ixu3)
    ixd = (ixd0, ixd1, ixd2, ixd3)
    rows = (rows0, rows1)
    semr = (semr0, semr1)
    NCH = EPT // CH
    srow = wid * NCH

    def fire_idx(g, s4):
        for h in range(H):
            pltpu.async_copy(idx4r.at[g * 4 + h], ixh[s4][h], semi)
        pltpu.async_copy(sdr.at[g], ixu[s4], semi)
        pltpu.async_copy(dstr.at[g], ixd[s4], semi)

    def wait_idx(s4):
        for h in range(H):
            pltpu.make_async_copy(idx4r.at[0], ixh[s4][h], semi).wait()
        pltpu.make_async_copy(sdr.at[0], ixu[s4], semi).wait()
        pltpu.make_async_copy(dstr.at[0], ixd[s4], semi).wait()

    def fire_rows(s4, b):
        for h in range(H):
            pltpu.async_copy(xw4.at[ixh[s4][h]], rows[b].at[h], semr[b])

    def wait_rows(s4, b):
        for h in range(H):
            pltpu.make_async_copy(xw4.at[ixh[s4][h]], rows[b].at[h],
                                  semr[b]).wait()

    def fire_uu(s4):
        pltpu.async_copy(xu.at[ixu[s4]], uu_v, semu)

    def wait_uu(s4):
        pltpu.make_async_copy(xu.at[ixu[s4]], uu_v, semu).wait()

    fire_idx(srow, 0)
    wait_idx(0)
    fire_rows(0, 0)
    fire_uu(0)
    fire_idx(srow + 1, 1)
    fire_idx(srow + 2, 2)

    def quad(Q, carry):
        for r in range(4):
            g = Q * 4 + r
            b = r & 1
            rv = rows[b]
            rn = (r + 1) & 3
            if r > 0:
                pltpu.make_async_copy(
                    msg_v, agg_sh.at[ixd[r - 1]], semw).wait()
            else:
                @pl.when(Q > 0)
                def _():
                    pltpu.make_async_copy(
                        msg_v, agg_sh.at[ixd[3]], semw).wait()

            @pl.when(g + 1 < NCH)
            def _():
                wait_idx(rn)
                fire_rows(rn, 1 - b)

            @pl.when(g + 3 < NCH)
            def _():
                fire_idx(srow + g + 3, (r + 3) & 3)

            wait_rows(r, b)
            wait_uu(r)
            cv = cvec_v[...]

            def edge(e, c3):
                l = uu_v[e, pl.ds(0, 16)] - uu_v[CH + e, pl.ds(0, 16)] + cv
                m = jnp.maximum(l, _lane_gather(l, rot1))
                m = jnp.maximum(m, _lane_gather(m, rot2))
                ex = jnp.exp(l - m)
                sm = ex + _lane_gather(ex, rot1)
                sm = sm + _lane_gather(sm, rot2)
                q = ex / sm
                qs = [_lane_bcast(q, h) for h in range(H)]
                for k in range(C // 16):
                    acc = qs[0] * rv[0, e, pl.ds(k * 16, 16)]
                    acc = acc + qs[1] * rv[1, e, pl.ds(k * 16, 16)]
                    acc = acc + qs[2] * rv[2, e, pl.ds(k * 16, 16)]
                    acc = acc + qs[3] * rv[3, e, pl.ds(k * 16, 16)]
                    msg_v[e, pl.ds(k * 16, 16)] = acc
                return c3

            lax.fori_loop(0, CH, edge, 0, unroll=4)

            @pl.when(g + 1 < NCH)
            def _():
                fire_uu(rn)

            pltpu.async_copy(msg_v, agg_sh.at[ixd[r]], semw, add=True)
        return carry

    lax.fori_loop(0, NCH // 4, quad, 0)
    pltpu.make_async_copy(msg_v, agg_sh.at[ixd[3]], semw).wait()
    plsc.subcore_barrier()
    pltpu.sync_copy(agg_sh.at[pl.ds(sid * AGGROW, AGGROW)],
                    out.at[cid, pl.ds(sid * AGGROW, AGGROW)])


@functools.partial(
    pl.kernel,
    out_type=jax.ShapeDtypeStruct((2, NPAD, C), jnp.float32),
    mesh=plsc.VectorSubcoreMesh(core_axis_name="c", subcore_axis_name="s"),
    compiler_params=pltpu.CompilerParams(needs_layout_passes=False),
    scratch_types=[
        pltpu.VMEM_SHARED((AGGR, C), jnp.float32),
        pltpu.VMEM((16,), jnp.float32),
        pltpu.VMEM((CH,), jnp.int32),
        pltpu.VMEM((CH,), jnp.int32),
        pltpu.VMEM((CH,), jnp.int32),
        pltpu.VMEM((CH,), jnp.int32),
        pltpu.VMEM((CH,), jnp.int32),
        pltpu.VMEM((CH,), jnp.int32),
        pltpu.VMEM((CH,), jnp.int32),
        pltpu.VMEM((CH,), jnp.int32),
        pltpu.VMEM((CH,), jnp.int32),
        pltpu.VMEM((CH,), jnp.int32),
        pltpu.VMEM((CH,), jnp.int32),
        pltpu.VMEM((CH,), jnp.int32),
        pltpu.VMEM((CH,), jnp.int32),
        pltpu.VMEM((CH,), jnp.int32),
        pltpu.VMEM((CH,), jnp.int32),
        pltpu.VMEM((CH,), jnp.int32),
        pltpu.VMEM((2 * CH,), jnp.int32),
        pltpu.VMEM((2 * CH,), jnp.int32),
        pltpu.VMEM((2 * CH,), jnp.int32),
        pltpu.VMEM((2 * CH,), jnp.int32),
        pltpu.VMEM((CH,), jnp.int32),
        pltpu.VMEM((CH,), jnp.int32),
        pltpu.VMEM((CH,), jnp.int32),
        pltpu.VMEM((CH,), jnp.int32),
        pltpu.VMEM((H, CH, C), jnp.float32),
        pltpu.VMEM((H, CH, C), jnp.float32),
        pltpu.VMEM((2 * CH, C), jnp.float32),
        pltpu.VMEM((CH, C), jnp.float32),
        pltpu.SemaphoreType.DMA,
        pltpu.SemaphoreType.DMA,
        pltpu.SemaphoreType.DMA,
        pltpu.SemaphoreType.DMA,
        pltpu.SemaphoreType.DMA,
    ],
)
def _sc_edge(xw4, xu, idx4r, sdr, dstr, cvec, zer, out, *rest):
    _edge_body(xw4, xu, idx4r, sdr, dstr, cvec, zer, out, *rest)


def _deg_body(dstr, onesr, zer, out, deg_sh, idx_d, ones_v, sem):
    cid = lax.axis_index("c")
    sid = lax.axis_index("s")
    wid = sid * 2 + cid
    pltpu.sync_copy(zer, deg_sh.at[pl.ds(sid * AGGROW, AGGROW)])
    pltpu.sync_copy(onesr, ones_v)
    plsc.subcore_barrier()

    def chunk(g, carry):
        base = wid * (EPT // CH) + g
        pltpu.sync_copy(dstr.at[base], idx_d)
        pltpu.sync_copy(ones_v, deg_sh.at[idx_d], add=True)
        return carry

    lax.fori_loop(0, EPT // CH, chunk, 0)
    plsc.subcore_barrier()
    pltpu.sync_copy(deg_sh.at[pl.ds(sid * AGGROW, AGGROW)],
                    out.at[cid, pl.ds(sid * AGGROW, AGGROW)])


@functools.partial(
    pl.kernel,
    out_type=jax.ShapeDtypeStruct((2, NPAD, C), jnp.float32),
    mesh=plsc.VectorSubcoreMesh(core_axis_name="c", subcore_axis_name="s"),
    compiler_params=pltpu.CompilerParams(needs_layout_passes=False),
    scratch_types=[
        pltpu.VMEM_SHARED((AGGR, C), jnp.float32),
        pltpu.VMEM((CH,), jnp.int32),
        pltpu.VMEM((CH, C), jnp.float32),
        pltpu.SemaphoreType.DMA,
    ],
)
def _sc_deg(dstr, onesr, zer, out, *rest):
    _deg_body(dstr, onesr, zer, out, *rest)


def kernel(verts, params, edges):
    src, dst = edges[0], edges[1]
    E = src.shape[0]
    x = jnp.pad(verts, ((0, NPAD - N), (0, 0)))

    srcp = jnp.concatenate([src, jnp.zeros((EP - E,), src.dtype)]).reshape(
        EP // CH, CH)
    dstp = jnp.concatenate([dst, jnp.full((EP - E,), N, dst.dtype)]).reshape(
        EP // CH, CH)
    s4 = srcp * 4
    idx4p = jnp.stack([s4, s4 + 1, s4 + 2, s4 + 3],
                      axis=1).reshape(EP // CH * 4, CH)
    sdp = jnp.concatenate([srcp, dstp], axis=1)

    zer128 = jnp.zeros((AGGROW, C), jnp.float32)
    ones128 = jnp.ones((CH, C), jnp.float32)

    deg2 = _sc_deg(dstp, ones128, zer128)

    xs = []
    for p in params['convs']:
        U128 = jnp.concatenate(
            [jnp.tile(p['U'], (1, 4)), jnp.zeros((C, C - 16), jnp.float32)],
            axis=1)
        cpad = jnp.full((1, C), NEG, jnp.float32).at[0, :H].set(p['c'])
        cvec = jnp.tile(p['c'], 4)
        brow = p['b'].reshape(1, C)
        xw, xu = _mm(x, p['W'], U128)
        xw4 = xw.reshape(NPAD * 4, C)
        agg2 = _sc_edge(xw4, xu, idx4p, sdp, dstp, cvec, zer128)
        x = _epilogue(agg2, xw, deg2, cpad, brow)
        xs.append(x[:N])

    wc = params['conv1d_w']
    cb8 = jnp.broadcast_to(params['conv1d_b'][:, None], (8 * C, 8))
    (W0, b0), (W1, b1), (W2, b2) = params['lins']
    w2p = jnp.pad(W2, ((0, 0), (0, C - W2.shape[1])))
    b2p = jnp.pad(b2, (0, C - b2.shape[0])).reshape(1, C)
    out = _head(xs[0], xs[1], xs[2],
                wc[:, 0:C], wc[:, C:2 * C], wc[:, 2 * C:3 * C], cb8,
                W0[:8 * C], W0[8 * C:], b0.reshape(1, 512),
                W1, b1.reshape(1, 256), w2p, b2p)
    return out[:, :10]


# speedup vs baseline: 1.1205x; 1.0968x over previous
import functools
import jax
import jax.numpy as jnp
from jax import lax
from jax.experimental import pallas as pl
from jax.experimental.pallas import tpu as pltpu
from jax.experimental.pallas import tpu_sc as plsc

H = 4
N = 10000
C = 128
NPAD = 10112
NB = 128
HB = 400
NEG = -1e30


def _mm_body(x_ref, w_ref, u_ref, b_ref, xw_ref, xu_ref):
    x = x_ref[...]
    xw_ref[...] = jnp.dot(x, w_ref[...],
                          preferred_element_type=jnp.float32) + b_ref[...]
    xu_ref[...] = jnp.dot(x, u_ref[...], preferred_element_type=jnp.float32)


def _mm(x, W640, U128, b640):
    nblk = NPAD // NB
    return pl.pallas_call(
        _mm_body,
        grid=(nblk,),
        in_specs=[pl.BlockSpec((NB, C), lambda i: (i, 0)),
                  pl.BlockSpec((C, 5 * C), lambda i: (0, 0)),
                  pl.BlockSpec((C, C), lambda i: (0, 0)),
                  pl.BlockSpec((1, 5 * C), lambda i: (0, 0))],
        out_specs=[pl.BlockSpec((NB, 5 * C), lambda i: (i, 0)),
                   pl.BlockSpec((NB, C), lambda i: (i, 0))],
        out_shape=[jax.ShapeDtypeStruct((NPAD, 5 * C), jnp.float32),
                   jax.ShapeDtypeStruct((NPAD, C), jnp.float32)],
    )(x, W640, U128, b640)


def _epi_body(agg_ref, xw_ref, deg_ref, cpad_ref, b_ref, out_ref):
    a = agg_ref[0] + agg_ref[1]
    deg = deg_ref[0] + deg_ref[1] + 1.0
    cp = cpad_ref[...]
    m = jnp.max(cp, axis=1, keepdims=True)
    e = jnp.exp(cp - m)
    q0 = e / jnp.sum(e, axis=1, keepdims=True)
    xw = xw_ref[...]
    sm = jnp.zeros_like(a)
    for h in range(H):
        qh = q0[0:1, h:h + 1]
        sm = sm + qh * xw[:, h * C:(h + 1) * C]
    y = (a + sm) / deg + b_ref[...]
    out_ref[...] = jnp.where(y > 0, y, jnp.exp(y) - 1.0)


def _epilogue(agg2, xw, deg2, cpad, brow):
    nblk = NPAD // NB
    return pl.pallas_call(
        _epi_body,
        grid=(nblk,),
        in_specs=[pl.BlockSpec((2, NB, C), lambda i: (0, i, 0)),
                  pl.BlockSpec((NB, 5 * C), lambda i: (i, 0)),
                  pl.BlockSpec((2, NB, C), lambda i: (0, i, 0)),
                  pl.BlockSpec((1, C), lambda i: (0, 0)),
                  pl.BlockSpec((1, C), lambda i: (0, 0))],
        out_specs=pl.BlockSpec((NB, C), lambda i: (i, 0)),
        out_shape=jax.ShapeDtypeStruct((NPAD, C), jnp.float32),
    )(agg2, xw, deg2, cpad, brow)


def _head_body(x1_ref, x2_ref, x3_ref, wc1, wc2, wc3, cb_ref,
               w0a, w0b, b0, w1, b1, w2, b2, out_ref, maxs, sums):
    i = pl.program_id(0)
    nblk = pl.num_programs(0)
    dn = (((1,), (1,)), ((), ()))
    y = lax.dot_general(wc1[...], x1_ref[...], dn, preferred_element_type=jnp.float32)
    y = y + lax.dot_general(wc2[...], x2_ref[...], dn, preferred_element_type=jnp.float32)
    y = y + lax.dot_general(wc3[...], x3_ref[...], dn, preferred_element_type=jnp.float32)
    y = y + cb_ref[:, 0:1]
    y = jnp.where(y >= 0, y, 0.2 * y)
    ymax = jnp.broadcast_to(jnp.max(y, axis=1, keepdims=True), (8 * C, C))
    ysum = jnp.broadcast_to(jnp.sum(y, axis=1, keepdims=True), (8 * C, C))

    @pl.when(i == 0)
    def _():
        maxs[...] = ymax
        sums[...] = ysum

    @pl.when(i > 0)
    def _():
        maxs[...] = jnp.maximum(maxs[...], ymax)
        sums[...] = sums[...] + ysum

    @pl.when(i == nblk - 1)
    def _():
        dc = (((0,), (0,)), ((), ()))
        z = lax.dot_general(maxs[...], w0a[...], dc, preferred_element_type=jnp.float32)
        z = z + lax.dot_general(sums[...] * (1.0 / N), w0b[...], dc,
                                preferred_element_type=jnp.float32)
        z = z + b0[...]
        z = jnp.where(z > 0, z, jnp.exp(z) - 1.0)
        z = jnp.dot(z, w1[...], preferred_element_type=jnp.float32) + b1[...]
        z = jnp.where(z > 0, z, jnp.exp(z) - 1.0)
        z = jnp.dot(z, w2[...], preferred_element_type=jnp.float32) + b2[...]
        out_ref[...] = jnp.tanh(z[0:1, :])


def _head(x1, x2, x3, wc1, wc2, wc3, cb8, w0a, w0b, b0, w1, b1, w2p, b2p):
    nblk = N // HB
    return pl.pallas_call(
        _head_body,
        grid=(nblk,),
        in_specs=[pl.BlockSpec((HB, C), lambda i: (i, 0)),
                  pl.BlockSpec((HB, C), lambda i: (i, 0)),
                  pl.BlockSpec((HB, C), lambda i: (i, 0)),
                  pl.BlockSpec((8 * C, C), lambda i: (0, 0)),
                  pl.BlockSpec((8 * C, C), lambda i: (0, 0)),
                  pl.BlockSpec((8 * C, C), lambda i: (0, 0)),
                  pl.BlockSpec((8 * C, 8), lambda i: (0, 0)),
                  pl.BlockSpec((8 * C, 512), lambda i: (0, 0)),
                  pl.BlockSpec((8 * C, 512), lambda i: (0, 0)),
                  pl.BlockSpec((1, 512), lambda i: (0, 0)),
                  pl.BlockSpec((512, 256), lambda i: (0, 0)),
                  pl.BlockSpec((1, 256), lambda i: (0, 0)),
                  pl.BlockSpec((256, C), lambda i: (0, 0)),
                  pl.BlockSpec((1, C), lambda i: (0, 0))],
        out_specs=pl.BlockSpec((1, C), lambda i: (0, 0)),
        out_shape=jax.ShapeDtypeStruct((1, C), jnp.float32),
        scratch_shapes=[pltpu.VMEM((8 * C, C), jnp.float32),
                        pltpu.VMEM((8 * C, C), jnp.float32)],
    )(x1, x2, x3, wc1, wc2, wc3, cb8, w0a, w0b, b0, w1, b1, w2p, b2p)


EP = 163840
EPT = EP // 32
CH = 32
AGGR = 10112
AGGROW = AGGR // 16
NROW = NPAD // 16


def _lane_gather(v, idx):
    dn = lax.GatherDimensionNumbers(offset_dims=(), collapsed_slice_dims=(0,),
                                    start_index_map=(0,))
    return lax.gather(v, idx[:, None], dn, (1,),
                      mode=lax.GatherScatterMode.PROMISE_IN_BOUNDS)


def _lane_bcast(v, t):
    return _lane_gather(v, jnp.full((16,), t, jnp.int32))


def _edge_body(xw, xu, srcr, dstr, zer, out,
               agg_sh, ixs0, ixs1, ixs2, ixs3, ixd0, ixd1, ixd2, ixd3,
               rows0, rows1, ud_v, msg_v, semr0, semr1, semu, semw, semi):
    cid = lax.axis_index("c")
    sid = lax.axis_index("s")
    wid = sid * 2 + cid
    pltpu.sync_copy(zer, agg_sh.at[pl.ds(sid * AGGROW, AGGROW)])
    plsc.subcore_barrier()

    lane = lax.iota(jnp.int32, 16)
    rot1 = jnp.bitwise_or(jnp.bitwise_and(lane, 12),
                          jnp.bitwise_and(lane + 1, 3))
    rot2 = jnp.bitwise_or(jnp.bitwise_and(lane, 12),
                          jnp.bitwise_and(lane + 2, 3))
    ixs = (ixs0, ixs1, ixs2, ixs3)
    ixd = (ixd0, ixd1, ixd2, ixd3)
    rows = (rows0, rows1)
    semr = (semr0, semr1)
    NCH = EPT // CH
    srow = wid * NCH

    def fire_idx(g, s4):
        pltpu.async_copy(srcr.at[g], ixs[s4], semi)
        pltpu.async_copy(dstr.at[g], ixd[s4], semi)

    def wait_idx(s4):
        pltpu.make_async_copy(srcr.at[0], ixs[s4], semi).wait()
        pltpu.make_async_copy(dstr.at[0], ixd[s4], semi).wait()

    fire_idx(srow, 0)
    wait_idx(0)
    pltpu.async_copy(xw.at[ixs0], rows0, semr0)
    pltpu.async_copy(xu.at[ixd0], ud_v, semu)
    fire_idx(srow + 1, 1)
    fire_idx(srow + 2, 2)

    def quad(Q, carry):
        for r in range(4):
            g = Q * 4 + r
            b = r & 1
            rv = rows[b]
            rn = (r + 1) & 3
            if r > 0:
                pltpu.make_async_copy(
                    msg_v, agg_sh.at[ixd[r - 1]], semw).wait()
            else:
                @pl.when(Q > 0)
                def _():
                    pltpu.make_async_copy(
                        msg_v, agg_sh.at[ixd[3]], semw).wait()

            @pl.when(g + 1 < NCH)
            def _():
                wait_idx(rn)
                pltpu.async_copy(xw.at[ixs[rn]], rows[1 - b], semr[1 - b])

            @pl.when(g + 3 < NCH)
            def _():
                fire_idx(srow + g + 3, (r + 3) & 3)

            pltpu.make_async_copy(xw.at[ixs[r]], rv, semr[b]).wait()
            pltpu.make_async_copy(xu.at[ixd[r]], ud_v, semu).wait()

            def edge(e, c3):
                l = rv[e, pl.ds(4 * C, 16)] - ud_v[e, pl.ds(0, 16)]
                m = jnp.maximum(l, _lane_gather(l, rot1))
                m = jnp.maximum(m, _lane_gather(m, rot2))
                ex = jnp.exp(l - m)
                sm = ex + _lane_gather(ex, rot1)
                sm = sm + _lane_gather(sm, rot2)
                q = ex / sm
                qs = [_lane_bcast(q, h) for h in range(H)]
                for k in range(C // 16):
                    acc = qs[0] * rv[e, pl.ds(k * 16, 16)]
                    acc = acc + qs[1] * rv[e, pl.ds(C + k * 16, 16)]
                    acc = acc + qs[2] * rv[e, pl.ds(2 * C + k * 16, 16)]
                    acc = acc + qs[3] * rv[e, pl.ds(3 * C + k * 16, 16)]
                    msg_v[e, pl.ds(k * 16, 16)] = acc
                return c3

            lax.fori_loop(0, CH, edge, 0, unroll=4)

            @pl.when(g + 1 < NCH)
            def _():
                pltpu.async_copy(xu.at[ixd[rn]], ud_v, semu)

            pltpu.async_copy(msg_v, agg_sh.at[ixd[r]], semw, add=True)
        return carry

    lax.fori_loop(0, NCH // 4, quad, 0)
    pltpu.make_async_copy(msg_v, agg_sh.at[ixd[3]], semw).wait()
    plsc.subcore_barrier()
    pltpu.sync_copy(agg_sh.at[pl.ds(sid * AGGROW, AGGROW)],
                    out.at[cid, pl.ds(sid * AGGROW, AGGROW)])


@functools.partial(
    pl.kernel,
    out_type=jax.ShapeDtypeStruct((2, NPAD, C), jnp.float32),
    mesh=plsc.VectorSubcoreMesh(core_axis_name="c", subcore_axis_name="s"),
    compiler_params=pltpu.CompilerParams(needs_layout_passes=False),
    scratch_types=[
        pltpu.VMEM_SHARED((AGGR, C), jnp.float32),
        pltpu.VMEM((CH,), jnp.int32),
        pltpu.VMEM((CH,), jnp.int32),
        pltpu.VMEM((CH,), jnp.int32),
        pltpu.VMEM((CH,), jnp.int32),
        pltpu.VMEM((CH,), jnp.int32),
        pltpu.VMEM((CH,), jnp.int32),
        pltpu.VMEM((CH,), jnp.int32),
        pltpu.VMEM((CH,), jnp.int32),
        pltpu.VMEM((CH, 5 * C), jnp.float32),
        pltpu.VMEM((CH, 5 * C), jnp.float32),
        pltpu.VMEM((CH, C), jnp.float32),
        pltpu.VMEM((CH, C), jnp.float32),
        pltpu.SemaphoreType.DMA,
        pltpu.SemaphoreType.DMA,
        pltpu.SemaphoreType.DMA,
        pltpu.SemaphoreType.DMA,
        pltpu.SemaphoreType.DMA,
    ],
)
def _sc_edge(xw, xu, srcr, dstr, zer, out, *rest):
    _edge_body(xw, xu, srcr, dstr, zer, out, *rest)


def _deg_body(dstr, onesr, zer, out, deg_sh, idx_d, ones_v, sem):
    cid = lax.axis_index("c")
    sid = lax.axis_index("s")
    wid = sid * 2 + cid
    pltpu.sync_copy(zer, deg_sh.at[pl.ds(sid * AGGROW, AGGROW)])
    pltpu.sync_copy(onesr, ones_v)
    plsc.subcore_barrier()

    def chunk(g, carry):
        base = wid * (EPT // CH) + g
        pltpu.sync_copy(dstr.at[base], idx_d)
        pltpu.sync_copy(ones_v, deg_sh.at[idx_d], add=True)
        return carry

    lax.fori_loop(0, EPT // CH, chunk, 0)
    plsc.subcore_barrier()
    pltpu.sync_copy(deg_sh.at[pl.ds(sid * AGGROW, AGGROW)],
                    out.at[cid, pl.ds(sid * AGGROW, AGGROW)])


@functools.partial(
    pl.kernel,
    out_type=jax.ShapeDtypeStruct((2, NPAD, C), jnp.float32),
    mesh=plsc.VectorSubcoreMesh(core_axis_name="c", subcore_axis_name="s"),
    compiler_params=pltpu.CompilerParams(needs_layout_passes=False),
    scratch_types=[
        pltpu.VMEM_SHARED((AGGR, C), jnp.float32),
        pltpu.VMEM((CH,), jnp.int32),
        pltpu.VMEM((CH, C), jnp.float32),
        pltpu.SemaphoreType.DMA,
    ],
)
def _sc_deg(dstr, onesr, zer, out, *rest):
    _deg_body(dstr, onesr, zer, out, *rest)


def kernel(verts, params, edges):
    src, dst = edges[0], edges[1]
    E = src.shape[0]
    x = jnp.pad(verts, ((0, NPAD - N), (0, 0)))

    srcp = jnp.concatenate([src, jnp.zeros((EP - E,), src.dtype)]).reshape(
        EP // CH, CH)
    dstp = jnp.concatenate([dst, jnp.full((EP - E,), N, dst.dtype)]).reshape(
        EP // CH, CH)

    zer128 = jnp.zeros((AGGROW, C), jnp.float32)
    ones128 = jnp.ones((CH, C), jnp.float32)

    deg2 = _sc_deg(dstp, ones128, zer128)

    xs = []
    for p in params['convs']:
        U16 = jnp.tile(p['U'], (1, 4))
        W640 = jnp.concatenate(
            [p['W'], U16, jnp.zeros((C, C - 16), jnp.float32)], axis=1)
        U128 = jnp.concatenate(
            [U16, jnp.zeros((C, C - 16), jnp.float32)], axis=1)
        cpad = jnp.full((1, C), NEG, jnp.float32).at[0, :H].set(p['c'])
        b640 = jnp.zeros((1, 5 * C), jnp.float32).at[0, 4 * C:4 * C + 16].set(
            jnp.tile(p['c'], 4))
        brow = p['b'].reshape(1, C)
        xw, xu = _mm(x, W640, U128, b640)
        agg2 = _sc_edge(xw, xu, srcp, dstp, zer128)
        x = _epilogue(agg2, xw, deg2, cpad, brow)
        xs.append(x[:N])

    wc = params['conv1d_w']
    cb8 = jnp.broadcast_to(params['conv1d_b'][:, None], (8 * C, 8))
    (W0, b0), (W1, b1), (W2, b2) = params['lins']
    w2p = jnp.pad(W2, ((0, 0), (0, C - W2.shape[1])))
    b2p = jnp.pad(b2, (0, C - b2.shape[0])).reshape(1, C)
    out = _head(xs[0], xs[1], xs[2],
                wc[:, 0:C], wc[:, C:2 * C], wc[:, 2 * C:3 * C], cb8,
                W0[:8 * C], W0[8 * C:], b0.reshape(1, 512),
                W1, b1.reshape(1, 256), w2p, b2p)
    return out[:, :10]
